# baseline (device time: 254889 ns/iter reference)
import jax
import jax.numpy as jnp
from jax import lax
from jax.experimental import pallas as pl
from jax.experimental.pallas import tpu as pltpu

N_DEV = 4
M_PER = 2048
K = 8192
N_PER = 1024
KB = 1024
NK = K // KB


def _gelu(y):
    c = 0.7978845608028654
    return 0.5 * y * (1.0 + jnp.tanh(c * (y + 0.044715 * y * y * y)))


def _cast_bf16(x):
    def cast_body(x_ref, o_ref):
        o_ref[...] = x_ref[...].astype(jnp.bfloat16)

    m, k = x.shape
    return pl.pallas_call(
        cast_body,
        grid=(k // 1024,),
        in_specs=[pl.BlockSpec((m, 1024), lambda i: (0, i))],
        out_specs=pl.BlockSpec((m, 1024), lambda i: (0, i)),
        out_shape=jax.ShapeDtypeStruct((m, k), jnp.bfloat16),
    )(x)


def kernel(x, w_mat):
    my = lax.axis_index("i")
    x = _cast_bf16(x)
    order = jnp.mod(my + jnp.array([2, 1, 3, 0], dtype=jnp.int32), N_DEV)

    def body(order_ref, x_ref, w_ref, out_ref,
             acc_ref, send_buf, recv_buf,
             send_sems, recv_sems, copy_sem, stage_sems):
        t = pl.program_id(0)
        k = pl.program_id(1)
        my_pos = lax.axis_index("i")
        j = order_ref[t]

        @pl.when(jnp.logical_and(t == 0, k == 0))
        def _():
            barrier_sem = pltpu.get_barrier_semaphore()
            for p in range(N_DEV):
                @pl.when(p != my_pos)
                def _():
                    pl.semaphore_signal(
                        barrier_sem, inc=1,
                        device_id=(p,), device_id_type=pl.DeviceIdType.MESH,
                    )
            pl.semaphore_wait(barrier_sem, N_DEV - 1)

        xb = x_ref[...]
        wb = w_ref[...].astype(jnp.bfloat16)

        @pl.when(k == 0)
        def _():
            acc_ref[...] = jnp.dot(xb, wb, preferred_element_type=jnp.float32)

        @pl.when(k > 0)
        def _():
            acc_ref[...] += jnp.dot(xb, wb, preferred_element_type=jnp.float32)

        @pl.when(jnp.logical_and(k == NK - 1, t < N_DEV - 1))
        def _():
            g = jnp.mod(t, 2)
            r = jnp.mod(my_pos - j - 1, N_DEV)

            @pl.when(t >= 2)
            def _():
                pltpu.make_async_remote_copy(
                    src_ref=send_buf.at[0],
                    dst_ref=recv_buf.at[0],
                    send_sem=send_sems.at[0],
                    recv_sem=recv_sems.at[0],
                    device_id=(my_pos,),
                    device_id_type=pl.DeviceIdType.MESH,
                ).wait_send()

            send_buf[g] = _gelu(acc_ref[...]).astype(jnp.bfloat16)
            pltpu.make_async_remote_copy(
                src_ref=send_buf.at[g],
                dst_ref=recv_buf.at[r],
                send_sem=send_sems.at[g],
                recv_sem=recv_sems.at[r],
                device_id=(j,),
                device_id_type=pl.DeviceIdType.MESH,
            ).start()

        @pl.when(jnp.logical_and(t == N_DEV - 1, k == NK - 1))
        def _():
            acc_ref[...] = _gelu(acc_ref[...])
            pltpu.make_async_copy(
                acc_ref, out_ref.at[pl.ds(my_pos * M_PER, M_PER)], copy_sem
            ).start()

            for r in range(N_DEV - 1):
                s = jnp.mod(my_pos + r + 1, N_DEV)
                pltpu.make_async_remote_copy(
                    src_ref=send_buf.at[r],
                    dst_ref=recv_buf.at[r],
                    send_sem=send_sems.at[r],
                    recv_sem=recv_sems.at[r],
                    device_id=(my_pos,),
                    device_id_type=pl.DeviceIdType.MESH,
                ).wait_recv()
                if r == 0:
                    pltpu.make_async_copy(
                        acc_ref,
                        out_ref.at[pl.ds(my_pos * M_PER, M_PER)],
                        copy_sem,
                    ).wait()
                else:
                    pltpu.make_async_copy(
                        acc_ref, out_ref.at[pl.ds(0, M_PER)], stage_sems
                    ).wait()
                acc_ref[...] = recv_buf[r].astype(jnp.float32)
                pltpu.make_async_copy(
                    acc_ref, out_ref.at[pl.ds(s * M_PER, M_PER)], stage_sems
                ).start()

            for g in range(2):
                pltpu.make_async_remote_copy(
                    src_ref=send_buf.at[g],
                    dst_ref=recv_buf.at[g],
                    send_sem=send_sems.at[g],
                    recv_sem=recv_sems.at[g],
                    device_id=(my_pos,),
                    device_id_type=pl.DeviceIdType.MESH,
                ).wait_send()
            pltpu.make_async_copy(
                acc_ref, out_ref.at[pl.ds(0, M_PER)], stage_sems
            ).wait()

    grid_spec = pltpu.PrefetchScalarGridSpec(
        num_scalar_prefetch=1,
        grid=(N_DEV, NK),
        in_specs=[
            pl.BlockSpec((M_PER, KB), lambda t, k, order: (0, k)),
            pl.BlockSpec((KB, N_PER), lambda t, k, order: (k, order[t])),
        ],
        out_specs=pl.BlockSpec(memory_space=pl.ANY),
        scratch_shapes=[
            pltpu.VMEM((M_PER, N_PER), jnp.float32),
            pltpu.VMEM((2, M_PER, N_PER), jnp.bfloat16),
            pltpu.VMEM((N_DEV - 1, M_PER, N_PER), jnp.bfloat16),
            pltpu.SemaphoreType.DMA((2,)),
            pltpu.SemaphoreType.DMA((N_DEV - 1,)),
            pltpu.SemaphoreType.DMA,
            pltpu.SemaphoreType.DMA,
        ],
    )
    return pl.pallas_call(
        body,
        grid_spec=grid_spec,
        out_shape=jax.ShapeDtypeStruct((N_DEV * M_PER, N_PER), jnp.float32),
        compiler_params=pltpu.CompilerParams(
            dimension_semantics=("arbitrary", "arbitrary"),
            collective_id=0,
            vmem_limit_bytes=65077248,
        ),
    )(order, x, w_mat)


# device time: 228728 ns/iter; 1.1144x vs baseline; 1.1144x over previous
import jax
import jax.numpy as jnp
from jax import lax
from jax.experimental import pallas as pl
from jax.experimental.pallas import tpu as pltpu

N_DEV = 4
M_PER = 2048
K = 8192
N_PER = 1024
KB = 1024
NK = K // KB


def _gelu(y):
    c = 0.7978845608028654
    return 0.5 * y * (1.0 + jnp.tanh(c * (y + 0.044715 * y * y * y)))


def kernel(x, w_mat):
    my = lax.axis_index("i")
    order = jnp.mod(my + jnp.array([2, 1, 3, 0], dtype=jnp.int32), N_DEV)

    def body(order_ref, x_ref, w_ref, out_ref,
             acc_ref, send_buf, recv_buf,
             send_sems, recv_sems, copy_sem, stage_sems):
        t = pl.program_id(0)
        k = pl.program_id(1)
        my_pos = lax.axis_index("i")
        j = order_ref[t]

        @pl.when(jnp.logical_and(t == 0, k == 0))
        def _():
            barrier_sem = pltpu.get_barrier_semaphore()
            for p in range(N_DEV):
                @pl.when(p != my_pos)
                def _():
                    pl.semaphore_signal(
                        barrier_sem, inc=1,
                        device_id=(p,), device_id_type=pl.DeviceIdType.MESH,
                    )
            pl.semaphore_wait(barrier_sem, N_DEV - 1)

        xb = x_ref[...].astype(jnp.bfloat16)
        wb = w_ref[...].astype(jnp.bfloat16)

        @pl.when(k == 0)
        def _():
            acc_ref[...] = jnp.dot(xb, wb, preferred_element_type=jnp.float32)

        @pl.when(k > 0)
        def _():
            acc_ref[...] += jnp.dot(xb, wb, preferred_element_type=jnp.float32)

        @pl.when(jnp.logical_and(k == NK - 1, t < N_DEV - 1))
        def _():
            g = jnp.mod(t, 2)
            r = jnp.mod(my_pos - j - 1, N_DEV)

            @pl.when(t >= 2)
            def _():
                pltpu.make_async_remote_copy(
                    src_ref=send_buf.at[0],
                    dst_ref=recv_buf.at[0],
                    send_sem=send_sems.at[0],
                    recv_sem=recv_sems.at[0],
                    device_id=(my_pos,),
                    device_id_type=pl.DeviceIdType.MESH,
                ).wait_send()

            send_buf[g] = _gelu(acc_ref[...]).astype(jnp.bfloat16)
            pltpu.make_async_remote_copy(
                src_ref=send_buf.at[g],
                dst_ref=recv_buf.at[r],
                send_sem=send_sems.at[g],
                recv_sem=recv_sems.at[r],
                device_id=(j,),
                device_id_type=pl.DeviceIdType.MESH,
            ).start()

        @pl.when(jnp.logical_and(t == N_DEV - 1, k == NK - 1))
        def _():
            acc_ref[...] = _gelu(acc_ref[...])
            pltpu.make_async_copy(
                acc_ref, out_ref.at[pl.ds(my_pos * M_PER, M_PER)], copy_sem
            ).start()

            for r in range(N_DEV - 1):
                s = jnp.mod(my_pos + r + 1, N_DEV)
                pltpu.make_async_remote_copy(
                    src_ref=send_buf.at[r],
                    dst_ref=recv_buf.at[r],
                    send_sem=send_sems.at[r],
                    recv_sem=recv_sems.at[r],
                    device_id=(my_pos,),
                    device_id_type=pl.DeviceIdType.MESH,
                ).wait_recv()
                if r == 0:
                    pltpu.make_async_copy(
                        acc_ref,
                        out_ref.at[pl.ds(my_pos * M_PER, M_PER)],
                        copy_sem,
                    ).wait()
                else:
                    pltpu.make_async_copy(
                        acc_ref, out_ref.at[pl.ds(0, M_PER)], stage_sems
                    ).wait()
                acc_ref[...] = recv_buf[r].astype(jnp.float32)
                pltpu.make_async_copy(
                    acc_ref, out_ref.at[pl.ds(s * M_PER, M_PER)], stage_sems
                ).start()

            for g in range(2):
                pltpu.make_async_remote_copy(
                    src_ref=send_buf.at[g],
                    dst_ref=recv_buf.at[g],
                    send_sem=send_sems.at[g],
                    recv_sem=recv_sems.at[g],
                    device_id=(my_pos,),
                    device_id_type=pl.DeviceIdType.MESH,
                ).wait_send()
            pltpu.make_async_copy(
                acc_ref, out_ref.at[pl.ds(0, M_PER)], stage_sems
            ).wait()

    grid_spec = pltpu.PrefetchScalarGridSpec(
        num_scalar_prefetch=1,
        grid=(N_DEV, NK),
        in_specs=[
            pl.BlockSpec((M_PER, KB), lambda t, k, order: (0, k)),
            pl.BlockSpec((KB, N_PER), lambda t, k, order: (k, order[t])),
        ],
        out_specs=pl.BlockSpec(memory_space=pl.ANY),
        scratch_shapes=[
            pltpu.VMEM((M_PER, N_PER), jnp.float32),
            pltpu.VMEM((2, M_PER, N_PER), jnp.bfloat16),
            pltpu.VMEM((N_DEV - 1, M_PER, N_PER), jnp.bfloat16),
            pltpu.SemaphoreType.DMA((2,)),
            pltpu.SemaphoreType.DMA((N_DEV - 1,)),
            pltpu.SemaphoreType.DMA,
            pltpu.SemaphoreType.DMA,
        ],
    )
    return pl.pallas_call(
        body,
        grid_spec=grid_spec,
        out_shape=jax.ShapeDtypeStruct((N_DEV * M_PER, N_PER), jnp.float32),
        compiler_params=pltpu.CompilerParams(
            dimension_semantics=("arbitrary", "arbitrary"),
            collective_id=0,
            vmem_limit_bytes=65077248,
        ),
    )(order, x, w_mat)
